# baseline (speedup 1.0000x reference)
import jax
import jax.numpy as jnp
from jax import lax
from jax.experimental import pallas as pl
from jax.experimental.pallas import tpu as pltpu
from jax.experimental.pallas import tpu_sc as plsc

NC = 2
NS = 16
NW = NC * NS
LANES = 16

K = 80
NCB = 25
NBK = 5
NCH = NCB * NBK
NR = 80
NPAD = NR * 128


def _mesh():
    return plsc.VectorSubcoreMesh(core_axis_name="c", subcore_axis_name="s")


def _sc_params():
    return pltpu.CompilerParams(needs_layout_passes=False)


def _node_mlp_body(z_ref, wm1_ref, bm1_ref, wm2_ref, bm2_ref, m_ref):
    zb = z_ref[...]
    h = lax.dot_general(zb, wm1_ref[...], (((1,), (1,)), ((), ())),
                        preferred_element_type=jnp.float32) + bm1_ref[...]
    h = jnp.maximum(h, 0.0)
    m_ref[...] = lax.dot_general(h, wm2_ref[...], (((1,), (1,)), ((), ())),
                                 preferred_element_type=jnp.float32) + bm2_ref[...]


SLC = NPAD // NS


NCB_A = NCH


def _sc1a_body(z_hbm, src3_hbm, dst3_hbm,
               zi_out, zj_out,
               srcv2, dstv2, zi_a, zj_a, zi_b, zj_b, sem1):
    cid = lax.axis_index("c")
    sid = lax.axis_index("s")
    wid = cid * NS + sid
    epw = NCH * K
    base0 = wid * epw

    def block(blk, carry0):
        pltpu.sync_copy(src3_hbm.at[wid, blk], srcv2)
        pltpu.sync_copy(dst3_hbm.at[wid, blk], dstv2)
        pltpu.async_copy(z_hbm.at[srcv2.at[0]], zi_a, sem1)
        pltpu.async_copy(z_hbm.at[dstv2.at[0]], zj_a, sem1)

        def step(czi, czj, nzi, nzj, ci):
            pltpu.make_async_copy(z_hbm.at[srcv2.at[ci]], czi, sem1).wait()
            pltpu.make_async_copy(z_hbm.at[dstv2.at[ci]], czj, sem1).wait()

            @pl.when(ci + 1 < NCB_A)
            def _():
                pltpu.async_copy(z_hbm.at[srcv2.at[ci + 1]], nzi, sem1)
                pltpu.async_copy(z_hbm.at[dstv2.at[ci + 1]], nzj, sem1)

            b = pl.multiple_of(base0 + (blk * NCB_A + ci) * K, 8)
            pltpu.sync_copy(czi, zi_out.at[pl.ds(b, K)])
            pltpu.sync_copy(czj, zj_out.at[pl.ds(b, K)])

        def chunk(ci, carry):
            @pl.when(ci % 2 == 0)
            def _():
                step(zi_a, zj_a, zi_b, zj_b, ci)

            @pl.when(ci % 2 == 1)
            def _():
                step(zi_b, zj_b, zi_a, zj_a, ci)

            return carry

        lax.fori_loop(0, NCB_A, chunk, 0)
        return carry0

    lax.fori_loop(0, 1, block, 0)


def _sc1b_body(m_hbm, src3_hbm, dst3_hbm, z2d_hbm, z1d_hbm,
               aggp_out, degp_out,
               srcv2, dstv2, m_a, m_b, deg_v, agg_sh, sem1):
    cid = lax.axis_index("c")
    sid = lax.axis_index("s")
    wid = cid * NS + sid

    pltpu.sync_copy(z1d_hbm, deg_v)

    @pl.when(sid == 0)
    def _():
        pltpu.sync_copy(z2d_hbm, agg_sh)

    plsc.subcore_barrier()

    ones16 = jnp.ones((LANES,), jnp.float32)

    def block(blk, carry0):
        pltpu.sync_copy(src3_hbm.at[wid, blk], srcv2)
        pltpu.sync_copy(dst3_hbm.at[wid, blk], dstv2)
        pltpu.async_copy(m_hbm.at[dstv2.at[0]], m_a, sem1)

        def step(cm, nm, ci):
            pltpu.make_async_copy(m_hbm.at[dstv2.at[ci]], cm, sem1).wait()

            @pl.when(ci + 1 < NCB)
            def _():
                pltpu.async_copy(m_hbm.at[dstv2.at[ci + 1]], nm, sem1)

            pltpu.sync_copy(cm, agg_sh.at[srcv2.at[ci]], add=True)
            for g in range(K // LANES):
                idx16 = srcv2[ci, pl.ds(g * LANES, LANES)]
                plsc.addupdate_scatter(deg_v, [idx16], ones16)

        def chunk(ci, carry):
            @pl.when(ci % 2 == 0)
            def _():
                step(m_a, m_b, ci)

            @pl.when(ci % 2 == 1)
            def _():
                step(m_b, m_a, ci)

            return carry

        lax.fori_loop(0, NCB, chunk, 0)
        return carry0

    lax.fori_loop(0, NBK, block, 0)

    pltpu.sync_copy(deg_v, degp_out.at[wid])
    plsc.subcore_barrier()

    @pl.when(sid == 0)
    def _():
        pltpu.sync_copy(agg_sh, aggp_out.at[cid])


def _edge_body(zi_ref, zj_ref, et2_ref, w1_ref, b1_ref, w2t_ref, b2_ref,
               temb_ref, ew_ref):
    bf = jnp.bfloat16
    zi = zi_ref[...]
    zj = zj_ref[...]
    ad = jnp.abs(zi - zj).astype(bf)
    zib = zi.astype(bf)
    zjb = zj.astype(bf)
    w1 = w1_ref[...]
    dn = (((1,), (1,)), ((), ()))
    u = lax.dot_general(zib, w1[:, 0:128], dn, preferred_element_type=jnp.float32)
    u = u + lax.dot_general(zjb, w1[:, 128:256], dn, preferred_element_type=jnp.float32)
    u = u + lax.dot_general(ad, w1[:, 256:384], dn, preferred_element_type=jnp.float32)
    t2 = lax.dot_general(temb_ref[...].astype(bf), w1[:, 384:392], dn,
                         preferred_element_type=jnp.float32)
    t01 = t2 + b1_ref[...]
    u = u + lax.dot_general(et2_ref[...], t01, (((1,), (0,)), ((), ())),
                            preferred_element_type=jnp.float32)
    h = jnp.maximum(u, 0.0)
    wl = lax.dot_general(h, w2t_ref[...], (((1,), (0,)), ((), ())),
                         preferred_element_type=jnp.float32) + b2_ref[0]
    ew_ref[...] = jnp.exp(jax.nn.sigmoid(wl))


def _sc2_body(ew_hbm, src_hbm, z1d_hbm, alpha_out,
              srcb, ewb, se_v, acc, tb, hist_sh, sum_sh):
    cid = lax.axis_index("c")
    sid = lax.axis_index("s")
    wid = cid * NS + sid
    E = ew_hbm.shape[0]
    eph = E // NS
    epw = E // NW

    hb = pl.multiple_of(sid * eph, 8)
    pltpu.sync_copy(src_hbm.at[pl.ds(hb, eph)], srcb)
    pltpu.sync_copy(ew_hbm.at[pl.ds(hb, eph)], ewb)
    pltpu.sync_copy(z1d_hbm, se_v)

    def hist(i, carry):
        sl = pl.ds(i * LANES, LANES)
        plsc.addupdate_scatter(se_v, [srcb[sl]], ewb[sl])
        return carry

    lax.fori_loop(0, eph // LANES, hist, 0)

    pltpu.sync_copy(se_v, hist_sh.at[sid])
    plsc.subcore_barrier()

    off = pl.multiple_of(sid * SLC, 8)
    pltpu.sync_copy(hist_sh.at[0, pl.ds(off, SLC)], acc)
    for t in range(1, NS):
        pltpu.sync_copy(hist_sh.at[t, pl.ds(off, SLC)], tb)

        def addloop(i, carry):
            sl = pl.ds(i * LANES, LANES)
            acc[sl] = acc[sl] + tb[sl]
            return carry

        lax.fori_loop(0, SLC // LANES, addloop, 0)
    pltpu.sync_copy(acc, sum_sh.at[pl.ds(off, SLC)])
    plsc.subcore_barrier()

    pltpu.sync_copy(sum_sh, se_v)
    ab = pl.multiple_of(wid * epw, 8)
    pltpu.sync_copy(src_hbm.at[pl.ds(ab, epw)], srcb.at[pl.ds(0, epw)])
    pltpu.sync_copy(ew_hbm.at[pl.ds(ab, epw)], ewb.at[pl.ds(0, epw)])

    def norm(i, carry):
        sl = pl.ds(i * LANES, LANES)
        se16 = plsc.load_gather(se_v, [srcb[sl]])
        ewb[sl] = ewb[sl] / (se16 + 1e-12)
        return carry

    lax.fori_loop(0, epw // LANES, norm, 0)
    pltpu.sync_copy(ewb.at[pl.ds(0, epw)], alpha_out.at[pl.ds(ab, epw)])


def _ln_body(z_ref, aggp_ref, degp_ref, lnw_ref, lnb_ref, out_ref):
    z = z_ref[...]
    agg = aggp_ref[0] + aggp_ref[1]
    deg = degp_ref[0]
    for t in range(1, NW):
        deg = deg + degp_ref[t]
    y = z + agg / (deg + 1e-12)
    mu = jnp.mean(y, axis=1, keepdims=True)
    yc = y - mu
    var = jnp.mean(yc * yc, axis=1, keepdims=True)
    out_ref[...] = yc / jnp.sqrt(var + 1e-5) * lnw_ref[...] + lnb_ref[...]


def kernel(z, edge_index, edge_type, type_emb, W1, b1, W2, b2,
           Wm1, bm1, Wm2, bm2, ln_w, ln_b):
    N, D = z.shape
    E = edge_index.shape[1]
    src = edge_index[0]
    dst = edge_index[1]
    ets = edge_type.astype(jnp.float32)
    et2 = jnp.stack([1.0 - ets, ets], axis=-1)
    z2d0 = jnp.zeros((N, D), jnp.float32)
    z1d0 = jnp.zeros((NPAD,), jnp.float32)

    f32 = jnp.float32
    BN = 1000
    BE = 1280

    m_nodes = pl.pallas_call(
        _node_mlp_body,
        grid=(N // BN,),
        in_specs=[
            pl.BlockSpec((BN, D), lambda i: (i, 0)),
            pl.BlockSpec((D, D), lambda i: (0, 0)),
            pl.BlockSpec((1, D), lambda i: (0, 0)),
            pl.BlockSpec((D, D), lambda i: (0, 0)),
            pl.BlockSpec((1, D), lambda i: (0, 0)),
        ],
        out_specs=pl.BlockSpec((BN, D), lambda i: (i, 0)),
        out_shape=jax.ShapeDtypeStruct((N, D), f32),
    )(z, Wm1, bm1.reshape(1, D), Wm2, bm2.reshape(1, D))

    sc1a = pl.kernel(
        _sc1a_body,
        out_type=(
            jax.ShapeDtypeStruct((E, D), f32),
            jax.ShapeDtypeStruct((E, D), f32),
        ),
        mesh=_mesh(),
        compiler_params=_sc_params(),
        scratch_types=(
            pltpu.VMEM((NCH, K), jnp.int32),
            pltpu.VMEM((NCH, K), jnp.int32),
            pltpu.VMEM((K, D), f32),
            pltpu.VMEM((K, D), f32),
            pltpu.VMEM((K, D), f32),
            pltpu.VMEM((K, D), f32),
            pltpu.SemaphoreType.DMA,
        ),
    )
    sc1b = pl.kernel(
        _sc1b_body,
        out_type=(
            jax.ShapeDtypeStruct((NC, N, D), f32),
            jax.ShapeDtypeStruct((NW, NPAD), f32),
        ),
        mesh=_mesh(),
        compiler_params=_sc_params(),
        scratch_types=(
            pltpu.VMEM((NCB, K), jnp.int32),
            pltpu.VMEM((NCB, K), jnp.int32),
            pltpu.VMEM((K, D), f32),
            pltpu.VMEM((K, D), f32),
            pltpu.VMEM((NPAD,), f32),
            pltpu.VMEM_SHARED((N, D), f32),
            pltpu.SemaphoreType.DMA,
        ),
    )
    src3 = src.reshape(NW, NBK, NCB, K)
    dst3 = dst.reshape(NW, NBK, NCB, K)
    src3a = src.reshape(NW, 1, NCH, K)
    dst3a = dst.reshape(NW, 1, NCH, K)
    zi_g, zj_g = sc1a(z, src3a, dst3a)

    W1p = W1.astype(jnp.bfloat16)
    ew = pl.pallas_call(
        _edge_body,
        grid=(E // BE,),
        in_specs=[
            pl.BlockSpec((BE, D), lambda i: (i, 0)),
            pl.BlockSpec((BE, D), lambda i: (i, 0)),
            pl.BlockSpec((BE, 2), lambda i: (i, 0)),
            pl.BlockSpec((D, 3 * D + 8), lambda i: (0, 0)),
            pl.BlockSpec((1, D), lambda i: (0, 0)),
            pl.BlockSpec((D, 1), lambda i: (0, 0)),
            pl.BlockSpec(memory_space=pltpu.SMEM),
            pl.BlockSpec((2, 8), lambda i: (0, 0)),
        ],
        out_specs=pl.BlockSpec((BE, 1), lambda i: (i, 0)),
        out_shape=jax.ShapeDtypeStruct((E, 1), f32),
    )(zi_g, zj_g, et2, W1p, b1.reshape(1, D), W2.reshape(D, 1), b2, type_emb)
    ew_flat = ew.reshape(E)
    aggp, degp = sc1b(m_nodes, src3, dst3, z2d0, z1d0)

    eph = E // NS
    sc2 = pl.kernel(
        _sc2_body,
        out_type=jax.ShapeDtypeStruct((E,), f32),
        mesh=_mesh(),
        compiler_params=_sc_params(),
        scratch_types=(
            pltpu.VMEM((eph,), jnp.int32),
            pltpu.VMEM((eph,), f32),
            pltpu.VMEM((NPAD,), f32),
            pltpu.VMEM((SLC,), f32),
            pltpu.VMEM((SLC,), f32),
            pltpu.VMEM_SHARED((NS, NPAD), f32),
            pltpu.VMEM_SHARED((NPAD,), f32),
        ),
    )
    alpha = sc2(ew_flat, src, z1d0)

    degp3 = degp[:, :N].reshape(NW, N, 1)
    out = pl.pallas_call(
        _ln_body,
        grid=(N // BN,),
        in_specs=[
            pl.BlockSpec((BN, D), lambda i: (i, 0)),
            pl.BlockSpec((NC, BN, D), lambda i: (0, i, 0)),
            pl.BlockSpec((NW, BN, 1), lambda i: (0, i, 0)),
            pl.BlockSpec((1, D), lambda i: (0, 0)),
            pl.BlockSpec((1, D), lambda i: (0, 0)),
        ],
        out_specs=pl.BlockSpec((BN, D), lambda i: (i, 0)),
        out_shape=jax.ShapeDtypeStruct((N, D), f32),
    )(z, aggp, degp3, ln_w.reshape(1, D), ln_b.reshape(1, D))

    return (out, alpha)


# speedup vs baseline: 1.0014x; 1.0014x over previous
import jax
import jax.numpy as jnp
from jax import lax
from jax.experimental import pallas as pl
from jax.experimental.pallas import tpu as pltpu
from jax.experimental.pallas import tpu_sc as plsc

NC = 2
NS = 16
NW = NC * NS
LANES = 16

K = 80
NCB = 25
NBK = 5
NCH = NCB * NBK
NR = 80
NPAD = NR * 128


def _mesh():
    return plsc.VectorSubcoreMesh(core_axis_name="c", subcore_axis_name="s")


def _sc_params():
    return pltpu.CompilerParams(needs_layout_passes=False)


def _node_mlp_body(z_ref, wm1_ref, bm1_ref, wm2_ref, bm2_ref, m_ref):
    zb = z_ref[...]
    h = lax.dot_general(zb, wm1_ref[...], (((1,), (1,)), ((), ())),
                        preferred_element_type=jnp.float32) + bm1_ref[...]
    h = jnp.maximum(h, 0.0)
    m_ref[...] = lax.dot_general(h, wm2_ref[...], (((1,), (1,)), ((), ())),
                                 preferred_element_type=jnp.float32) + bm2_ref[...]


SLC = NPAD // NS


def _sc1a_body(z_hbm, src3_hbm, dst3_hbm,
               zi_out, zj_out,
               srcv2, dstv2, zi_a, zj_a, zi_b, zj_b, sem1):
    cid = lax.axis_index("c")
    sid = lax.axis_index("s")
    wid = cid * NS + sid
    epw = NCH * K
    base0 = wid * epw

    def block(blk, carry0):
        pltpu.sync_copy(src3_hbm.at[wid, blk], srcv2)
        pltpu.sync_copy(dst3_hbm.at[wid, blk], dstv2)
        pltpu.async_copy(z_hbm.at[srcv2.at[0]], zi_a, sem1)
        pltpu.async_copy(z_hbm.at[dstv2.at[0]], zj_a, sem1)

        def step(czi, czj, nzi, nzj, ci):
            pltpu.make_async_copy(z_hbm.at[srcv2.at[ci]], czi, sem1).wait()
            pltpu.make_async_copy(z_hbm.at[dstv2.at[ci]], czj, sem1).wait()

            @pl.when(ci + 1 < NCB)
            def _():
                pltpu.async_copy(z_hbm.at[srcv2.at[ci + 1]], nzi, sem1)
                pltpu.async_copy(z_hbm.at[dstv2.at[ci + 1]], nzj, sem1)

            b = pl.multiple_of(base0 + (blk * NCB + ci) * K, 8)
            pltpu.sync_copy(czi, zi_out.at[pl.ds(b, K)])
            pltpu.sync_copy(czj, zj_out.at[pl.ds(b, K)])

        def chunk(ci, carry):
            @pl.when(ci % 2 == 0)
            def _():
                step(zi_a, zj_a, zi_b, zj_b, ci)

            @pl.when(ci % 2 == 1)
            def _():
                step(zi_b, zj_b, zi_a, zj_a, ci)

            return carry

        lax.fori_loop(0, NCB, chunk, 0)
        return carry0

    lax.fori_loop(0, NBK, block, 0)


def _sc1b_body(m_hbm, src3_hbm, dst3_hbm, z2d_hbm, z1d_hbm,
               aggp_out, degp_out,
               srcv2, dstv2, m_a, m_b, deg_v, agg_sh, sem1):
    cid = lax.axis_index("c")
    sid = lax.axis_index("s")
    wid = cid * NS + sid

    pltpu.sync_copy(z1d_hbm, deg_v)

    @pl.when(sid == 0)
    def _():
        pltpu.sync_copy(z2d_hbm, agg_sh)

    plsc.subcore_barrier()

    ones16 = jnp.ones((LANES,), jnp.float32)

    def block(blk, carry0):
        pltpu.sync_copy(src3_hbm.at[wid, blk], srcv2)
        pltpu.sync_copy(dst3_hbm.at[wid, blk], dstv2)
        pltpu.async_copy(m_hbm.at[dstv2.at[0]], m_a, sem1)

        def step(cm, nm, ci):
            pltpu.make_async_copy(m_hbm.at[dstv2.at[ci]], cm, sem1).wait()

            @pl.when(ci + 1 < NCB)
            def _():
                pltpu.async_copy(m_hbm.at[dstv2.at[ci + 1]], nm, sem1)

            pltpu.sync_copy(cm, agg_sh.at[srcv2.at[ci]], add=True)
            for g in range(K // LANES):
                idx16 = srcv2[ci, pl.ds(g * LANES, LANES)]
                plsc.addupdate_scatter(deg_v, [idx16], ones16)

        def chunk(ci, carry):
            @pl.when(ci % 2 == 0)
            def _():
                step(m_a, m_b, ci)

            @pl.when(ci % 2 == 1)
            def _():
                step(m_b, m_a, ci)

            return carry

        lax.fori_loop(0, NCB, chunk, 0)
        return carry0

    lax.fori_loop(0, NBK, block, 0)

    pltpu.sync_copy(deg_v, degp_out.at[wid])
    plsc.subcore_barrier()

    @pl.when(sid == 0)
    def _():
        pltpu.sync_copy(agg_sh, aggp_out.at[cid])


def _edge_body(zi_ref, zj_ref, et2_ref, w1_ref, b1_ref, w2t_ref, b2_ref,
               temb_ref, ew_ref):
    bf = jnp.bfloat16
    zi = zi_ref[...]
    zj = zj_ref[...]
    ad = jnp.abs(zi - zj).astype(bf)
    zib = zi.astype(bf)
    zjb = zj.astype(bf)
    w1 = w1_ref[...]
    dn = (((1,), (1,)), ((), ()))
    u = lax.dot_general(zib, w1[:, 0:128], dn, preferred_element_type=jnp.float32)
    u = u + lax.dot_general(zjb, w1[:, 128:256], dn, preferred_element_type=jnp.float32)
    u = u + lax.dot_general(ad, w1[:, 256:384], dn, preferred_element_type=jnp.float32)
    t2 = lax.dot_general(temb_ref[...].astype(bf), w1[:, 384:392], dn,
                         preferred_element_type=jnp.float32)
    t01 = t2 + b1_ref[...]
    u = u + lax.dot_general(et2_ref[...], t01, (((1,), (0,)), ((), ())),
                            preferred_element_type=jnp.float32)
    h = jnp.maximum(u, 0.0)
    wl = lax.dot_general(h, w2t_ref[...], (((1,), (0,)), ((), ())),
                         preferred_element_type=jnp.float32) + b2_ref[0]
    ew_ref[...] = jnp.exp(jax.nn.sigmoid(wl))


def _sc2_body(ew_hbm, src_hbm, z1d_hbm, alpha_out,
              srcb, ewb, se_v, acc, tb, hist_sh, sum_sh):
    cid = lax.axis_index("c")
    sid = lax.axis_index("s")
    wid = cid * NS + sid
    E = ew_hbm.shape[0]
    eph = E // NS
    epw = E // NW

    hb = pl.multiple_of(sid * eph, 8)
    pltpu.sync_copy(src_hbm.at[pl.ds(hb, eph)], srcb)
    pltpu.sync_copy(ew_hbm.at[pl.ds(hb, eph)], ewb)
    pltpu.sync_copy(z1d_hbm, se_v)

    def hist(i, carry):
        sl = pl.ds(i * LANES, LANES)
        plsc.addupdate_scatter(se_v, [srcb[sl]], ewb[sl])
        return carry

    lax.fori_loop(0, eph // LANES, hist, 0)

    pltpu.sync_copy(se_v, hist_sh.at[sid])
    plsc.subcore_barrier()

    off = pl.multiple_of(sid * SLC, 8)
    pltpu.sync_copy(hist_sh.at[0, pl.ds(off, SLC)], acc)
    for t in range(1, NS):
        pltpu.sync_copy(hist_sh.at[t, pl.ds(off, SLC)], tb)

        def addloop(i, carry):
            sl = pl.ds(i * LANES, LANES)
            acc[sl] = acc[sl] + tb[sl]
            return carry

        lax.fori_loop(0, SLC // LANES, addloop, 0)
    pltpu.sync_copy(acc, sum_sh.at[pl.ds(off, SLC)])
    plsc.subcore_barrier()

    pltpu.sync_copy(sum_sh, se_v)
    ab = pl.multiple_of(wid * epw, 8)
    pltpu.sync_copy(src_hbm.at[pl.ds(ab, epw)], srcb.at[pl.ds(0, epw)])
    pltpu.sync_copy(ew_hbm.at[pl.ds(ab, epw)], ewb.at[pl.ds(0, epw)])

    def norm(i, carry):
        sl = pl.ds(i * LANES, LANES)
        se16 = plsc.load_gather(se_v, [srcb[sl]])
        ewb[sl] = ewb[sl] / (se16 + 1e-12)
        return carry

    lax.fori_loop(0, epw // LANES, norm, 0)
    pltpu.sync_copy(ewb.at[pl.ds(0, epw)], alpha_out.at[pl.ds(ab, epw)])


def _ln_body(z_ref, aggp_ref, degp_ref, lnw_ref, lnb_ref, out_ref):
    z = z_ref[...]
    agg = aggp_ref[0] + aggp_ref[1]
    deg = degp_ref[0]
    for t in range(1, NW):
        deg = deg + degp_ref[t]
    y = z + agg / (deg + 1e-12)
    mu = jnp.mean(y, axis=1, keepdims=True)
    yc = y - mu
    var = jnp.mean(yc * yc, axis=1, keepdims=True)
    out_ref[...] = yc / jnp.sqrt(var + 1e-5) * lnw_ref[...] + lnb_ref[...]


def kernel(z, edge_index, edge_type, type_emb, W1, b1, W2, b2,
           Wm1, bm1, Wm2, bm2, ln_w, ln_b):
    N, D = z.shape
    E = edge_index.shape[1]
    src = edge_index[0]
    dst = edge_index[1]
    ets = edge_type.astype(jnp.float32)
    et2 = jnp.stack([1.0 - ets, ets], axis=-1)
    z2d0 = jnp.zeros((N, D), jnp.float32)
    z1d0 = jnp.zeros((NPAD,), jnp.float32)

    f32 = jnp.float32
    BN = 1000
    BE = 1280

    m_nodes = pl.pallas_call(
        _node_mlp_body,
        grid=(N // BN,),
        in_specs=[
            pl.BlockSpec((BN, D), lambda i: (i, 0)),
            pl.BlockSpec((D, D), lambda i: (0, 0)),
            pl.BlockSpec((1, D), lambda i: (0, 0)),
            pl.BlockSpec((D, D), lambda i: (0, 0)),
            pl.BlockSpec((1, D), lambda i: (0, 0)),
        ],
        out_specs=pl.BlockSpec((BN, D), lambda i: (i, 0)),
        out_shape=jax.ShapeDtypeStruct((N, D), f32),
    )(z, Wm1, bm1.reshape(1, D), Wm2, bm2.reshape(1, D))

    sc1a = pl.kernel(
        _sc1a_body,
        out_type=(
            jax.ShapeDtypeStruct((E, D), f32),
            jax.ShapeDtypeStruct((E, D), f32),
        ),
        mesh=_mesh(),
        compiler_params=_sc_params(),
        scratch_types=(
            pltpu.VMEM((NCB, K), jnp.int32),
            pltpu.VMEM((NCB, K), jnp.int32),
            pltpu.VMEM((K, D), f32),
            pltpu.VMEM((K, D), f32),
            pltpu.VMEM((K, D), f32),
            pltpu.VMEM((K, D), f32),
            pltpu.SemaphoreType.DMA,
        ),
    )
    sc1b = pl.kernel(
        _sc1b_body,
        out_type=(
            jax.ShapeDtypeStruct((NC, N, D), f32),
            jax.ShapeDtypeStruct((NW, NPAD), f32),
        ),
        mesh=_mesh(),
        compiler_params=_sc_params(),
        scratch_types=(
            pltpu.VMEM((NCB, K), jnp.int32),
            pltpu.VMEM((NCB, K), jnp.int32),
            pltpu.VMEM((K, D), f32),
            pltpu.VMEM((K, D), f32),
            pltpu.VMEM((NPAD,), f32),
            pltpu.VMEM_SHARED((N, D), f32),
            pltpu.SemaphoreType.DMA,
        ),
    )
    src3 = src.reshape(NW, NBK, NCB, K)
    dst3 = dst.reshape(NW, NBK, NCB, K)
    zi_g, zj_g = sc1a(z, src3, dst3)
    aggp, degp = sc1b(m_nodes, src3, dst3, z2d0, z1d0)

    W1p = W1.astype(jnp.bfloat16)
    ew = pl.pallas_call(
        _edge_body,
        grid=(E // BE,),
        in_specs=[
            pl.BlockSpec((BE, D), lambda i: (i, 0)),
            pl.BlockSpec((BE, D), lambda i: (i, 0)),
            pl.BlockSpec((BE, 2), lambda i: (i, 0)),
            pl.BlockSpec((D, 3 * D + 8), lambda i: (0, 0)),
            pl.BlockSpec((1, D), lambda i: (0, 0)),
            pl.BlockSpec((D, 1), lambda i: (0, 0)),
            pl.BlockSpec(memory_space=pltpu.SMEM),
            pl.BlockSpec((2, 8), lambda i: (0, 0)),
        ],
        out_specs=pl.BlockSpec((BE, 1), lambda i: (i, 0)),
        out_shape=jax.ShapeDtypeStruct((E, 1), f32),
    )(zi_g, zj_g, et2, W1p, b1.reshape(1, D), W2.reshape(D, 1), b2, type_emb)
    ew_flat = ew.reshape(E)

    eph = E // NS
    sc2 = pl.kernel(
        _sc2_body,
        out_type=jax.ShapeDtypeStruct((E,), f32),
        mesh=_mesh(),
        compiler_params=_sc_params(),
        scratch_types=(
            pltpu.VMEM((eph,), jnp.int32),
            pltpu.VMEM((eph,), f32),
            pltpu.VMEM((NPAD,), f32),
            pltpu.VMEM((SLC,), f32),
            pltpu.VMEM((SLC,), f32),
            pltpu.VMEM_SHARED((NS, NPAD), f32),
            pltpu.VMEM_SHARED((NPAD,), f32),
        ),
    )
    alpha = sc2(ew_flat, src, z1d0)

    degp3 = degp[:, :N].reshape(NW, N, 1)
    out = pl.pallas_call(
        _ln_body,
        grid=(N // BN,),
        in_specs=[
            pl.BlockSpec((BN, D), lambda i: (i, 0)),
            pl.BlockSpec((NC, BN, D), lambda i: (0, i, 0)),
            pl.BlockSpec((NW, BN, 1), lambda i: (0, i, 0)),
            pl.BlockSpec((1, D), lambda i: (0, 0)),
            pl.BlockSpec((1, D), lambda i: (0, 0)),
        ],
        out_specs=pl.BlockSpec((BN, D), lambda i: (i, 0)),
        out_shape=jax.ShapeDtypeStruct((N, D), f32),
    )(z, aggp, degp3, ln_w.reshape(1, D), ln_b.reshape(1, D))

    return (out, alpha)
